# Initial kernel scaffold; baseline (speedup 1.0000x reference)
import jax
import jax.numpy as jnp
from jax.experimental import pallas as pl


def kernel(x, token_type, token_table, proj_W, proj_b, pos_table, type_table, gamma, beta):
    raise NotImplementedError("write your pallas kernel here")


# speedup vs baseline: 2.1088x; 2.1088x over previous
import functools
import math

import jax
import jax.numpy as jnp
from jax import lax
from jax.experimental import pallas as pl
from jax.experimental.pallas import tpu as pltpu
from jax.experimental.pallas import tpu_sc as plsc

_B = 64
_S = 512
_N = _B * _S
_D4 = 32
_D = 128

_NC = 2
_NS = 16
_NW = _NC * _NS
_B_PER_W = _N // _NW
_CHUNK = 128
_NCHUNK = _B_PER_W // _CHUNK

_ROWS_PER_TC_BLOCK = 8


def _sc_gather_kernel(table_hbm, idx_hbm, out_hbm, idx_v, rows_v, sem):
    wid = lax.axis_index("s") * _NC + lax.axis_index("c")
    base = wid * _B_PER_W
    pltpu.sync_copy(idx_hbm.at[pl.ds(base, _B_PER_W)], idx_v)
    copies = []
    for j in range(_NCHUNK):
        copies.append(
            pltpu.async_copy(
                table_hbm.at[idx_v.at[pl.ds(j * _CHUNK, _CHUNK)]],
                rows_v.at[pl.ds(j * _CHUNK, _CHUNK)],
                sem,
            )
        )
    for c in copies:
        c.wait()
    pltpu.sync_copy(rows_v, out_hbm.at[pl.ds(base, _B_PER_W)])


def _sc_gather(token_table, idx_flat):
    mesh = plsc.VectorSubcoreMesh(core_axis_name="c", subcore_axis_name="s")
    k = pl.kernel(
        _sc_gather_kernel,
        out_type=jax.ShapeDtypeStruct((_N, _D4), jnp.float32),
        mesh=mesh,
        compiler_params=pltpu.CompilerParams(use_tc_tiling_on_sc=False),
        scratch_types=[
            pltpu.VMEM((_B_PER_W,), jnp.int32),
            pltpu.VMEM((_B_PER_W, _D4), jnp.float32),
            pltpu.SemaphoreType.DMA,
        ],
    )
    return k(token_table, idx_flat)


def _tc_body(e_ref, tt_ref, w_ref, b_ref, pos_ref, type_ref, g_ref, bt_ref,
             out_ref):
    e = e_ref[...]
    h = 0.5 * e * (1.0 + lax.erf(e * (1.0 / math.sqrt(2.0))))
    r = e.shape[0]
    h2 = jnp.dot(
        h.reshape(r * _S, _D4), w_ref[...],
        preferred_element_type=jnp.float32,
    ).reshape(r, _S, _D)
    h2 = h2 + b_ref[...].reshape(1, 1, _D)
    h2 = h2 + pos_ref[...][None, :, :]
    t0 = type_ref[0, :]
    t1 = type_ref[1, :]
    tt = tt_ref[...].astype(jnp.float32)
    h2 = h2 + t0[None, None, :] + tt[:, :, None] * (t1 - t0)[None, None, :]
    mean = jnp.mean(h2, axis=-1, keepdims=True)
    d = h2 - mean
    var = jnp.mean(d * d, axis=-1, keepdims=True)
    out_ref[...] = (d * lax.rsqrt(var + 1e-12)) * g_ref[...].reshape(1, 1, _D) \
        + bt_ref[...].reshape(1, 1, _D)


def _tc_compute(gathered, token_type, proj_W, proj_b, pos, type_table, gamma,
                beta):
    r = _ROWS_PER_TC_BLOCK
    grid = (_B // r,)
    return pl.pallas_call(
        _tc_body,
        grid=grid,
        in_specs=[
            pl.BlockSpec((r, _S, _D4), lambda i: (i, 0, 0)),
            pl.BlockSpec((r, _S), lambda i: (i, 0)),
            pl.BlockSpec((_D4, _D), lambda i: (0, 0)),
            pl.BlockSpec((1, _D), lambda i: (0, 0)),
            pl.BlockSpec((_S, _D), lambda i: (0, 0)),
            pl.BlockSpec((2, _D), lambda i: (0, 0)),
            pl.BlockSpec((1, _D), lambda i: (0, 0)),
            pl.BlockSpec((1, _D), lambda i: (0, 0)),
        ],
        out_specs=pl.BlockSpec((r, _S, _D), lambda i: (i, 0, 0)),
        out_shape=jax.ShapeDtypeStruct((_B, _S, _D), jnp.float32),
    )(gathered, token_type, proj_W, proj_b, pos, type_table, gamma, beta)


def kernel(x, token_type, token_table, proj_W, proj_b, pos_table, type_table,
           gamma, beta):
    idx_flat = x.reshape(_N)
    gathered = _sc_gather(token_table, idx_flat)
    return _tc_compute(
        gathered.reshape(_B, _S, _D4),
        token_type,
        proj_W,
        proj_b.reshape(1, _D),
        pos_table[:_S],
        type_table,
        gamma.reshape(1, _D),
        beta.reshape(1, _D),
    )


# speedup vs baseline: 2.1099x; 1.0006x over previous
import functools
import math

import jax
import jax.numpy as jnp
from jax import lax
from jax.experimental import pallas as pl
from jax.experimental.pallas import tpu as pltpu
from jax.experimental.pallas import tpu_sc as plsc

_B = 64
_S = 512
_N = _B * _S
_D4 = 32
_D = 128
_PACK = _D // _D4
_VP = 100000 // _PACK

_NC = 2
_NS = 16
_NW = _NC * _NS
_B_PER_W = _N // _NW
_CHUNK = 128
_NCHUNK = _B_PER_W // _CHUNK

_ROWS_PER_TC_BLOCK = 8


def _sc_gather_kernel(table_hbm, idx_hbm, out_hbm, idx_v, buf_v, sem):
    wid = lax.axis_index("s") * _NC + lax.axis_index("c")
    base = wid * _B_PER_W
    pltpu.sync_copy(idx_hbm.at[pl.ds(base, _B_PER_W)], idx_v)
    copies = []
    for j in range(_NCHUNK):
        copies.append(
            pltpu.async_copy(
                table_hbm.at[idx_v.at[pl.ds(j * _CHUNK, _CHUNK)]],
                buf_v.at[j % 2],
                sem,
            )
        )
        if j > 0:
            copies[j - 1].wait()
            pltpu.sync_copy(
                buf_v.at[(j - 1) % 2],
                out_hbm.at[pl.ds(base + (j - 1) * _CHUNK, _CHUNK)],
            )
    copies[_NCHUNK - 1].wait()
    pltpu.sync_copy(
        buf_v.at[(_NCHUNK - 1) % 2],
        out_hbm.at[pl.ds(base + (_NCHUNK - 1) * _CHUNK, _CHUNK)],
    )


def _sc_gather(table4, idx4):
    mesh = plsc.VectorSubcoreMesh(core_axis_name="c", subcore_axis_name="s")
    k = pl.kernel(
        _sc_gather_kernel,
        out_type=jax.ShapeDtypeStruct((_N, _D), jnp.float32),
        mesh=mesh,
        scratch_types=[
            pltpu.VMEM((_B_PER_W,), jnp.int32),
            pltpu.VMEM((2, _CHUNK, _D), jnp.float32),
            pltpu.SemaphoreType.DMA,
        ],
    )
    return k(table4, idx4)


def _tc_body(e_ref, m_ref, w_ref, b_ref, pos_ref, type_ref, g_ref, bt_ref,
             out_ref):
    e = e_ref[...]
    r = e.shape[0]
    mq = m_ref[...]
    lane_q = lax.broadcasted_iota(jnp.int32, (r, _S, _D), 2) // _D4
    mask = lane_q == (mq & 3)[:, :, None]
    h = 0.5 * e * (1.0 + lax.erf(e * (1.0 / math.sqrt(2.0))))
    h = jnp.where(mask, h, 0.0)
    w = w_ref[...]
    w4 = jnp.concatenate([w, w, w, w], axis=0)
    h2 = jnp.dot(
        h.reshape(r * _S, _D), w4,
        preferred_element_type=jnp.float32,
    ).reshape(r, _S, _D)
    h2 = h2 + b_ref[...].reshape(1, 1, _D)
    h2 = h2 + pos_ref[...][None, :, :]
    t0 = type_ref[0, :]
    t1 = type_ref[1, :]
    tt = (mq >> 2).astype(jnp.float32)
    h2 = h2 + t0[None, None, :] + tt[:, :, None] * (t1 - t0)[None, None, :]
    mean = jnp.mean(h2, axis=-1, keepdims=True)
    d = h2 - mean
    var = jnp.mean(d * d, axis=-1, keepdims=True)
    out_ref[...] = (d * lax.rsqrt(var + 1e-12)) * g_ref[...].reshape(1, 1, _D) \
        + bt_ref[...].reshape(1, 1, _D)


def _tc_compute(gathered, mq, proj_W, proj_b, pos, type_table, gamma, beta):
    r = _ROWS_PER_TC_BLOCK
    grid = (_B // r,)
    return pl.pallas_call(
        _tc_body,
        grid=grid,
        in_specs=[
            pl.BlockSpec((r, _S, _D), lambda i: (i, 0, 0)),
            pl.BlockSpec((r, _S), lambda i: (i, 0)),
            pl.BlockSpec((_D4, _D), lambda i: (0, 0)),
            pl.BlockSpec((1, _D), lambda i: (0, 0)),
            pl.BlockSpec((_S, _D), lambda i: (0, 0)),
            pl.BlockSpec((2, _D), lambda i: (0, 0)),
            pl.BlockSpec((1, _D), lambda i: (0, 0)),
            pl.BlockSpec((1, _D), lambda i: (0, 0)),
        ],
        out_specs=pl.BlockSpec((r, _S, _D), lambda i: (i, 0, 0)),
        out_shape=jax.ShapeDtypeStruct((_B, _S, _D), jnp.float32),
    )(gathered, mq, proj_W, proj_b, pos, type_table, gamma, beta)


def kernel(x, token_type, token_table, proj_W, proj_b, pos_table, type_table,
           gamma, beta):
    table4 = token_table.reshape(_VP, _D)
    idx_flat = x.reshape(_N)
    idx4 = idx_flat >> 2
    mq = (x & 3) | (token_type << 2)
    gathered = _sc_gather(table4, idx4)
    return _tc_compute(
        gathered.reshape(_B, _S, _D),
        mq,
        proj_W,
        proj_b.reshape(1, _D),
        pos_table[:_S],
        type_table,
        gamma.reshape(1, _D),
        beta.reshape(1, _D),
    )
